# baseline (speedup 1.0000x reference)
import jax
import jax.numpy as jnp
from jax import lax
from jax.experimental import pallas as pl
from jax.experimental.pallas import tpu as pltpu
from jax.experimental.pallas import tpu_sc as plsc

N = 10000
E = 320000
D = 128
NCLS = 10
NC, NS, LANES = 2, 16, 16
NW = NC * NS
CHUNK = 128
NROWS = E // CHUNK
NR = NROWS // NW
NEXTRA = NROWS - NR * NW

_f32 = jnp.float32
_i32 = jnp.int32


def _mesh():
    return plsc.VectorSubcoreMesh(
        core_axis_name="c", subcore_axis_name="s", num_cores=NC, num_subcores=NS
    )


def _row_of(w, i):
    return jnp.where(i == NR, NW * NR + w, w * NR + i)


def _load_my_indices(src3_hbm, dst3_hbm, sbuf, dbuf, w):
    pltpu.sync_copy(src3_hbm.at[pl.ds(w * NR, NR)], sbuf.at[pl.ds(0, NR)])
    pltpu.sync_copy(dst3_hbm.at[pl.ds(w * NR, NR)], dbuf.at[pl.ds(0, NR)])

    @pl.when(w < NEXTRA)
    def _():
        pltpu.sync_copy(src3_hbm.at[pl.ds(NW * NR + w, 1)], sbuf.at[pl.ds(NR, 1)])
        pltpu.sync_copy(dst3_hbm.at[pl.ds(NW * NR + w, 1)], dbuf.at[pl.ds(NR, 1)])


_DEG_GROUP = 6
_DEG_NGROUPS = NR // _DEG_GROUP


def _deg_body(src3_hbm, dst3_hbm, out_hbm, sbuf, dbuf, ones_v, zer_v,
              deg_s, deg_d, sem):
    cid = lax.axis_index("c")
    sid = lax.axis_index("s")
    w = cid * NS + sid
    for j in range(CHUNK // LANES):
        ones_v[pl.ds(j * LANES, LANES)] = jnp.full((LANES,), 1.0, _f32)
    for j in range(1024 // LANES):
        zer_v[pl.ds(j * LANES, LANES)] = jnp.zeros((LANES,), _f32)

    @pl.when(sid < 10)
    def _():
        pltpu.sync_copy(zer_v.at[pl.ds(0, 1000)], deg_s.at[pl.ds(sid * 1000, 1000)])
        pltpu.sync_copy(zer_v.at[pl.ds(0, 1000)], deg_d.at[pl.ds(sid * 1000, 1000)])

    _load_my_indices(src3_hbm, dst3_hbm, sbuf, dbuf, w)
    plsc.subcore_barrier()

    def group(g, carry):
        descs = []
        for j in range(_DEG_GROUP):
            i = g * _DEG_GROUP + j
            descs.append(pltpu.async_copy(ones_v, deg_s.at[sbuf.at[i, 0]], sem, add=True))
            descs.append(pltpu.async_copy(ones_v, deg_d.at[dbuf.at[i, 0]], sem, add=True))
        for d in descs:
            d.wait()
        return carry

    lax.fori_loop(0, _DEG_NGROUPS, group, 0)

    @pl.when(w < NEXTRA)
    def _():
        d1 = pltpu.async_copy(ones_v, deg_s.at[sbuf.at[NR, 0]], sem, add=True)
        d2 = pltpu.async_copy(ones_v, deg_d.at[dbuf.at[NR, 0]], sem, add=True)
        d1.wait()
        d2.wait()

    plsc.subcore_barrier()

    @pl.when(sid == 0)
    def _():
        pltpu.sync_copy(deg_s, out_hbm.at[cid, 0])
        pltpu.sync_copy(deg_d, out_hbm.at[cid, 2])


def _deg(src3, dst3):
    call = pl.kernel(
        _deg_body,
        out_type=jax.ShapeDtypeStruct((NC, 4, N), _f32),
        mesh=_mesh(),
        scratch_types=[
            pltpu.VMEM((NR + 1, 1, CHUNK), _i32),
            pltpu.VMEM((NR + 1, 1, CHUNK), _i32),
            pltpu.VMEM((CHUNK,), _f32),
            pltpu.VMEM((1024,), _f32),
            pltpu.VMEM_SHARED((N,), _f32),
            pltpu.VMEM_SHARED((N,), _f32),
            pltpu.SemaphoreType.DMA,
        ],
    )
    return call(src3, dst3)


_NPAIRS = NR // 2


def _agg_body(xs_hbm, src3_hbm, dst3_hbm, zeros_hbm, out_hbm,
              dbuf, sidx0, sidx1, rows0, rows1, agg_sh, sem_g, sem_s, sem_i):
    cid = lax.axis_index("c")
    sid = lax.axis_index("s")
    w = cid * NS + sid
    nch = jnp.where(w < NEXTRA, NR + 1, NR)

    @pl.when(sid < 10)
    def _():
        pltpu.sync_copy(zeros_hbm.at[pl.ds(sid * 1000, 1000)],
                        agg_sh.at[pl.ds(sid * 1000, 1000)])

    pltpu.sync_copy(dst3_hbm.at[pl.ds(w * NR, NR)], dbuf.at[pl.ds(0, NR)])

    @pl.when(w < NEXTRA)
    def _():
        pltpu.sync_copy(dst3_hbm.at[pl.ds(NW * NR + w, 1)], dbuf.at[pl.ds(NR, 1)])

    plsc.subcore_barrier()

    def sload(i, sidx):
        pltpu.async_copy(src3_hbm.at[pl.ds(_row_of(w, i), 1)], sidx, sem_i)

    def iwait(sidx):
        pltpu.make_async_copy(src3_hbm.at[pl.ds(0, 1)], sidx, sem_i).wait()

    def gather(sidx, rows):
        pltpu.async_copy(xs_hbm.at[sidx.at[0, 0]], rows, sem_g)

    def gwait(rows):
        pltpu.make_async_copy(xs_hbm.at[sidx0.at[0, 0]], rows, sem_g).wait()

    def scatter(i, rows):
        pltpu.async_copy(rows, agg_sh.at[dbuf.at[i, 0]], sem_s, add=True)

    def swait(rows):
        pltpu.make_async_copy(rows, agg_sh.at[dbuf.at[0, 0]], sem_s).wait()

    sload(0, sidx0)
    iwait(sidx0)
    gather(sidx0, rows0)
    sload(1, sidx1)
    iwait(sidx1)

    def pair(i2, carry):
        i = 2 * i2
        gwait(rows0)

        @pl.when(i2 > 0)
        def _():
            swait(rows1)

        gather(sidx1, rows1)

        @pl.when(i + 2 < nch)
        def _():
            sload(i + 2, sidx0)

        scatter(i, rows0)
        gwait(rows1)
        swait(rows0)

        @pl.when(i + 2 < nch)
        def _():
            iwait(sidx0)
            gather(sidx0, rows0)

        @pl.when(i + 3 < nch)
        def _():
            sload(i + 3, sidx1)

        scatter(i + 1, rows1)

        @pl.when(i + 3 < nch)
        def _():
            iwait(sidx1)

        return carry

    lax.fori_loop(0, _NPAIRS, pair, 0)
    swait(rows1)

    @pl.when(nch == NR + 1)
    def _():
        gwait(rows0)
        scatter(NR, rows0)
        swait(rows0)

    plsc.subcore_barrier()

    @pl.when(sid < 10)
    def _():
        pltpu.sync_copy(agg_sh.at[pl.ds(sid * 1000, 1000)],
                        out_hbm.at[cid, pl.ds(sid * 1000, 1000)])


def _agg(xs, src3, dst3, zeros):
    call = pl.kernel(
        _agg_body,
        out_type=jax.ShapeDtypeStruct((NC, N, D), _f32),
        mesh=_mesh(),
        scratch_types=[
            pltpu.VMEM((NR + 1, 1, CHUNK), _i32),
            pltpu.VMEM((1, 1, CHUNK), _i32),
            pltpu.VMEM((1, 1, CHUNK), _i32),
            pltpu.VMEM((CHUNK, D), _f32),
            pltpu.VMEM((CHUNK, D), _f32),
            pltpu.VMEM_SHARED((N, D), _f32),
            pltpu.SemaphoreType.DMA,
            pltpu.SemaphoreType.DMA,
            pltpu.SemaphoreType.DMA,
        ],
    )
    return call(xs, src3, dst3, zeros)


def _prep(degs, h):
    def body(degs_ref, h_ref, hs_ref, ns_ref, nd_ref):
        d = degs_ref[...]
        ns = lax.rsqrt(jnp.maximum(d[0, 0] + d[1, 0], 1.0))[:, None]
        nd = lax.rsqrt(jnp.maximum(d[0, 2] + d[1, 2], 1.0))[:, None]
        hs_ref[...] = h_ref[...] * ns
        ns_ref[...] = ns
        nd_ref[...] = nd

    return pl.pallas_call(
        body,
        out_shape=[
            jax.ShapeDtypeStruct((N, D), _f32),
            jax.ShapeDtypeStruct((N, 1), _f32),
            jax.ShapeDtypeStruct((N, 1), _f32),
        ],
    )(degs, h)


_GRID = 10
_BR = N // _GRID


def _layer(agg, nd, ns, W, b):
    def body(agg_ref, nd_ref, ns_ref, W_ref, b_ref, o_ref):
        a = (agg_ref[0] + agg_ref[1]) * nd_ref[...]
        y = jnp.dot(a, W_ref[...], preferred_element_type=_f32) + b_ref[...]
        o_ref[...] = jnp.maximum(y, 0.0) * ns_ref[...]

    return pl.pallas_call(
        body,
        grid=(_GRID,),
        in_specs=[
            pl.BlockSpec((NC, _BR, D), lambda i: (0, i, 0)),
            pl.BlockSpec((_BR, 1), lambda i: (i, 0)),
            pl.BlockSpec((_BR, 1), lambda i: (i, 0)),
            pl.BlockSpec((D, D), lambda i: (0, 0)),
            pl.BlockSpec((1, D), lambda i: (0, 0)),
        ],
        out_specs=pl.BlockSpec((_BR, D), lambda i: (i, 0)),
        out_shape=jax.ShapeDtypeStruct((N, D), _f32),
    )(agg, nd, ns, W, b.reshape(1, D))


def _out(agg, nd, W2, b2, Wc_p, bc_p):
    def body(agg_ref, nd_ref, W_ref, b_ref, Wc_ref, bc_ref, o_ref):
        a = (agg_ref[0] + agg_ref[1]) * nd_ref[...]
        y = jnp.dot(a, W_ref[...], preferred_element_type=_f32) + b_ref[...]
        y = jnp.maximum(y, 0.0)
        o_ref[...] = jnp.dot(y, Wc_ref[...], preferred_element_type=_f32) + bc_ref[...]

    return pl.pallas_call(
        body,
        grid=(_GRID,),
        in_specs=[
            pl.BlockSpec((NC, _BR, D), lambda i: (0, i, 0)),
            pl.BlockSpec((_BR, 1), lambda i: (i, 0)),
            pl.BlockSpec((D, D), lambda i: (0, 0)),
            pl.BlockSpec((1, D), lambda i: (0, 0)),
            pl.BlockSpec((D, D), lambda i: (0, 0)),
            pl.BlockSpec((1, D), lambda i: (0, 0)),
        ],
        out_specs=pl.BlockSpec((_BR, D), lambda i: (i, 0)),
        out_shape=jax.ShapeDtypeStruct((N, D), _f32),
    )(agg, nd, W2, b2.reshape(1, D), Wc_p, bc_p.reshape(1, D))


def kernel(h, edge_index, W1, b1, W2, b2, Wc, bc):
    src3 = edge_index[0].astype(_i32).reshape(NROWS, 1, CHUNK)
    dst3 = edge_index[1].astype(_i32).reshape(NROWS, 1, CHUNK)
    zeros = jnp.zeros((N, D), _f32)

    degs = _deg(src3, dst3)
    hs1, ns, nd = _prep(degs, h)
    agg1 = _agg(hs1, src3, dst3, zeros)
    hs2 = _layer(agg1, nd, ns, W1, b1)
    agg2 = _agg(hs2, src3, dst3, zeros)

    Wc_p = jnp.zeros((D, D), _f32).at[:, :NCLS].set(Wc)
    bc_p = jnp.zeros((D,), _f32).at[:NCLS].set(bc)
    outp = _out(agg2, nd, W2, b2, Wc_p, bc_p)
    return outp[:, :NCLS]


# speedup vs baseline: 1.0159x; 1.0159x over previous
import jax
import jax.numpy as jnp
from jax import lax
from jax.experimental import pallas as pl
from jax.experimental.pallas import tpu as pltpu
from jax.experimental.pallas import tpu_sc as plsc

N = 10000
E = 320000
D = 128
NCLS = 10
NC, NS, LANES = 2, 16, 16
NW = NC * NS
CHUNK = 128
NROWS = E // CHUNK
NR = NROWS // NW
NEXTRA = NROWS - NR * NW

_f32 = jnp.float32
_i32 = jnp.int32


def _mesh():
    return plsc.VectorSubcoreMesh(
        core_axis_name="c", subcore_axis_name="s", num_cores=NC, num_subcores=NS
    )


def _row_of(w, i):
    return jnp.where(i == NR, NW * NR + w, w * NR + i)


_DEG_GROUP = 6
_DEG_NGROUPS = NR // _DEG_GROUP


def _deg_body(src3_hbm, dst3_hbm, out_hbm, sbuf, dbuf, ones_v, zer_v,
              deg_s, deg_d, sem):
    cid = lax.axis_index("c")
    sid = lax.axis_index("s")
    w = cid * NS + sid
    for j in range(CHUNK // LANES):
        ones_v[pl.ds(j * LANES, LANES)] = jnp.full((LANES,), 1.0, _f32)
    for j in range(1024 // LANES):
        zer_v[pl.ds(j * LANES, LANES)] = jnp.zeros((LANES,), _f32)

    @pl.when(sid < 10)
    def _():
        pltpu.sync_copy(zer_v.at[pl.ds(0, 1000)], deg_s.at[pl.ds(sid * 1000, 1000)])
        pltpu.sync_copy(zer_v.at[pl.ds(0, 1000)], deg_d.at[pl.ds(sid * 1000, 1000)])

    pltpu.sync_copy(src3_hbm.at[pl.ds(w * NR, NR)], sbuf.at[pl.ds(0, NR)])
    pltpu.sync_copy(dst3_hbm.at[pl.ds(w * NR, NR)], dbuf.at[pl.ds(0, NR)])

    @pl.when(w < NEXTRA)
    def _():
        pltpu.sync_copy(src3_hbm.at[pl.ds(NW * NR + w, 1)], sbuf.at[pl.ds(NR, 1)])
        pltpu.sync_copy(dst3_hbm.at[pl.ds(NW * NR + w, 1)], dbuf.at[pl.ds(NR, 1)])

    plsc.subcore_barrier()

    def group(g, carry):
        descs = []
        for j in range(_DEG_GROUP):
            i = g * _DEG_GROUP + j
            descs.append(pltpu.async_copy(ones_v, deg_s.at[sbuf.at[i, 0]], sem, add=True))
            descs.append(pltpu.async_copy(ones_v, deg_d.at[dbuf.at[i, 0]], sem, add=True))
        for d in descs:
            d.wait()
        return carry

    lax.fori_loop(0, _DEG_NGROUPS, group, 0)

    @pl.when(w < NEXTRA)
    def _():
        d1 = pltpu.async_copy(ones_v, deg_s.at[sbuf.at[NR, 0]], sem, add=True)
        d2 = pltpu.async_copy(ones_v, deg_d.at[dbuf.at[NR, 0]], sem, add=True)
        d1.wait()
        d2.wait()

    plsc.subcore_barrier()

    @pl.when(sid == 0)
    def _():
        pltpu.sync_copy(deg_s, out_hbm.at[cid, 0])
        pltpu.sync_copy(deg_d, out_hbm.at[cid, 2])


def _deg(src3, dst3):
    call = pl.kernel(
        _deg_body,
        out_type=jax.ShapeDtypeStruct((NC, 4, N), _f32),
        mesh=_mesh(),
        scratch_types=[
            pltpu.VMEM((NR + 1, 1, CHUNK), _i32),
            pltpu.VMEM((NR + 1, 1, CHUNK), _i32),
            pltpu.VMEM((CHUNK,), _f32),
            pltpu.VMEM((1024,), _f32),
            pltpu.VMEM_SHARED((N,), _f32),
            pltpu.VMEM_SHARED((N,), _f32),
            pltpu.SemaphoreType.DMA,
        ],
    )
    return call(src3, dst3)


_NPAIRS = NR // 2


def _agg_body(xs_hbm, src3_hbm, dst3_hbm, zeros_hbm, out_hbm,
              dbuf, sidx0, sidx1, rows0, rows1, agg_sh, sem):
    cid = lax.axis_index("c")
    sid = lax.axis_index("s")
    w = cid * NS + sid
    nch = jnp.where(w < NEXTRA, NR + 1, NR)

    @pl.when(sid < 10)
    def _():
        pltpu.sync_copy(zeros_hbm.at[pl.ds(sid * 1000, 1000)],
                        agg_sh.at[pl.ds(sid * 1000, 1000)])

    pltpu.sync_copy(dst3_hbm.at[pl.ds(w * NR, NR)], dbuf.at[pl.ds(0, NR)])

    @pl.when(w < NEXTRA)
    def _():
        pltpu.sync_copy(dst3_hbm.at[pl.ds(NW * NR + w, 1)], dbuf.at[pl.ds(NR, 1)])

    plsc.subcore_barrier()

    def sload(i, sidx):
        pltpu.sync_copy(src3_hbm.at[pl.ds(_row_of(w, i), 1)], sidx)

    def gather(sidx, rows):
        return pltpu.async_copy(xs_hbm.at[sidx.at[0, 0]], rows, sem)

    def gwait(rows):
        pltpu.make_async_copy(xs_hbm.at[sidx0.at[0, 0]], rows, sem).wait()

    def scatter(i, rows):
        pltpu.sync_copy(rows, agg_sh.at[dbuf.at[i, 0]], add=True)

    sload(0, sidx0)
    gather(sidx0, rows0)
    sload(1, sidx1)

    def pair(i2, carry):
        i = 2 * i2
        gwait(rows0)
        gather(sidx1, rows1)

        @pl.when(i + 2 < nch)
        def _():
            sload(i + 2, sidx0)

        scatter(i, rows0)
        gwait(rows1)

        @pl.when(i + 2 < nch)
        def _():
            gather(sidx0, rows0)

        @pl.when(i + 3 < nch)
        def _():
            sload(i + 3, sidx1)

        scatter(i + 1, rows1)
        return carry

    lax.fori_loop(0, _NPAIRS, pair, 0)

    @pl.when(nch == NR + 1)
    def _():
        gwait(rows0)
        scatter(NR, rows0)

    plsc.subcore_barrier()

    @pl.when(sid < 10)
    def _():
        pltpu.sync_copy(agg_sh.at[pl.ds(sid * 1000, 1000)],
                        out_hbm.at[cid, pl.ds(sid * 1000, 1000)])


def _agg(xs, src3, dst3, zeros):
    call = pl.kernel(
        _agg_body,
        out_type=jax.ShapeDtypeStruct((NC, N, D), _f32),
        mesh=_mesh(),
        scratch_types=[
            pltpu.VMEM((NR + 1, 1, CHUNK), _i32),
            pltpu.VMEM((1, 1, CHUNK), _i32),
            pltpu.VMEM((1, 1, CHUNK), _i32),
            pltpu.VMEM((CHUNK, D), _f32),
            pltpu.VMEM((CHUNK, D), _f32),
            pltpu.VMEM_SHARED((N, D), _f32),
            pltpu.SemaphoreType.DMA,
        ],
    )
    return call(xs, src3, dst3, zeros)


def _prep(degs, h):
    def body(degs_ref, h_ref, hs_ref, ns_ref, nd_ref):
        d = degs_ref[...]
        ns = lax.rsqrt(jnp.maximum(d[0, 0] + d[1, 0], 1.0))[:, None]
        nd = lax.rsqrt(jnp.maximum(d[0, 2] + d[1, 2], 1.0))[:, None]
        hs_ref[...] = h_ref[...] * ns
        ns_ref[...] = ns
        nd_ref[...] = nd

    return pl.pallas_call(
        body,
        out_shape=[
            jax.ShapeDtypeStruct((N, D), _f32),
            jax.ShapeDtypeStruct((N, 1), _f32),
            jax.ShapeDtypeStruct((N, 1), _f32),
        ],
    )(degs, h)


_GRID = 10
_BR = N // _GRID


def _layer(agg, nd, ns, W, b):
    def body(agg_ref, nd_ref, ns_ref, W_ref, b_ref, o_ref):
        a = (agg_ref[0] + agg_ref[1]) * nd_ref[...]
        y = jnp.dot(a, W_ref[...], preferred_element_type=_f32) + b_ref[...]
        o_ref[...] = jnp.maximum(y, 0.0) * ns_ref[...]

    return pl.pallas_call(
        body,
        grid=(_GRID,),
        in_specs=[
            pl.BlockSpec((NC, _BR, D), lambda i: (0, i, 0)),
            pl.BlockSpec((_BR, 1), lambda i: (i, 0)),
            pl.BlockSpec((_BR, 1), lambda i: (i, 0)),
            pl.BlockSpec((D, D), lambda i: (0, 0)),
            pl.BlockSpec((1, D), lambda i: (0, 0)),
        ],
        out_specs=pl.BlockSpec((_BR, D), lambda i: (i, 0)),
        out_shape=jax.ShapeDtypeStruct((N, D), _f32),
    )(agg, nd, ns, W, b.reshape(1, D))


def _out(agg, nd, W2, b2, Wc_p, bc_p):
    def body(agg_ref, nd_ref, W_ref, b_ref, Wc_ref, bc_ref, o_ref):
        a = (agg_ref[0] + agg_ref[1]) * nd_ref[...]
        y = jnp.dot(a, W_ref[...], preferred_element_type=_f32) + b_ref[...]
        y = jnp.maximum(y, 0.0)
        o_ref[...] = jnp.dot(y, Wc_ref[...], preferred_element_type=_f32) + bc_ref[...]

    return pl.pallas_call(
        body,
        grid=(_GRID,),
        in_specs=[
            pl.BlockSpec((NC, _BR, D), lambda i: (0, i, 0)),
            pl.BlockSpec((_BR, 1), lambda i: (i, 0)),
            pl.BlockSpec((D, D), lambda i: (0, 0)),
            pl.BlockSpec((1, D), lambda i: (0, 0)),
            pl.BlockSpec((D, D), lambda i: (0, 0)),
            pl.BlockSpec((1, D), lambda i: (0, 0)),
        ],
        out_specs=pl.BlockSpec((_BR, D), lambda i: (i, 0)),
        out_shape=jax.ShapeDtypeStruct((N, D), _f32),
    )(agg, nd, W2, b2.reshape(1, D), Wc_p, bc_p.reshape(1, D))


def kernel(h, edge_index, W1, b1, W2, b2, Wc, bc):
    src3 = edge_index[0].astype(_i32).reshape(NROWS, 1, CHUNK)
    dst3 = edge_index[1].astype(_i32).reshape(NROWS, 1, CHUNK)
    zeros = jnp.zeros((N, D), _f32)

    degs = _deg(src3, dst3)
    hs1, ns, nd = _prep(degs, h)
    agg1 = _agg(hs1, src3, dst3, zeros)
    hs2 = _layer(agg1, nd, ns, W1, b1)
    agg2 = _agg(hs2, src3, dst3, zeros)

    Wc_p = jnp.zeros((D, D), _f32).at[:, :NCLS].set(Wc)
    bc_p = jnp.zeros((D,), _f32).at[:NCLS].set(bc)
    outp = _out(agg2, nd, W2, b2, Wc_p, bc_p)
    return outp[:, :NCLS]


# speedup vs baseline: 1.0266x; 1.0105x over previous
import jax
import jax.numpy as jnp
from jax import lax
from jax.experimental import pallas as pl
from jax.experimental.pallas import tpu as pltpu
from jax.experimental.pallas import tpu_sc as plsc

N = 10000
E = 320000
D = 128
NCLS = 10
NC, NS, LANES = 2, 16, 16
NW = NC * NS
CHUNK = 128
NROWS = E // CHUNK
NR = NROWS // NW
NEXTRA = NROWS - NR * NW

_f32 = jnp.float32
_i32 = jnp.int32


def _mesh():
    return plsc.VectorSubcoreMesh(
        core_axis_name="c", subcore_axis_name="s", num_cores=NC, num_subcores=NS
    )


def _row_of(w, i):
    return jnp.where(i == NR, NW * NR + w, w * NR + i)


_DEG_GROUP = 6
_DEG_NGROUPS = NR // _DEG_GROUP


def _deg_body(src3_hbm, dst3_hbm, out_hbm, sbuf, dbuf, ones_v, zer_v,
              deg_s, deg_d, sem):
    cid = lax.axis_index("c")
    sid = lax.axis_index("s")
    w = cid * NS + sid
    for j in range(CHUNK // LANES):
        ones_v[pl.ds(j * LANES, LANES)] = jnp.full((LANES,), 1.0, _f32)
    for j in range(1024 // LANES):
        zer_v[pl.ds(j * LANES, LANES)] = jnp.zeros((LANES,), _f32)

    @pl.when(sid < 10)
    def _():
        pltpu.sync_copy(zer_v.at[pl.ds(0, 1000)], deg_s.at[pl.ds(sid * 1000, 1000)])
        pltpu.sync_copy(zer_v.at[pl.ds(0, 1000)], deg_d.at[pl.ds(sid * 1000, 1000)])

    pltpu.sync_copy(src3_hbm.at[pl.ds(w * NR, NR)], sbuf.at[pl.ds(0, NR)])
    pltpu.sync_copy(dst3_hbm.at[pl.ds(w * NR, NR)], dbuf.at[pl.ds(0, NR)])

    @pl.when(w < NEXTRA)
    def _():
        pltpu.sync_copy(src3_hbm.at[pl.ds(NW * NR + w, 1)], sbuf.at[pl.ds(NR, 1)])
        pltpu.sync_copy(dst3_hbm.at[pl.ds(NW * NR + w, 1)], dbuf.at[pl.ds(NR, 1)])

    plsc.subcore_barrier()

    def group(g, carry):
        descs = []
        for j in range(_DEG_GROUP):
            i = g * _DEG_GROUP + j
            descs.append(pltpu.async_copy(ones_v, deg_s.at[sbuf.at[i, 0]], sem, add=True))
            descs.append(pltpu.async_copy(ones_v, deg_d.at[dbuf.at[i, 0]], sem, add=True))
        for d in descs:
            d.wait()
        return carry

    lax.fori_loop(0, _DEG_NGROUPS, group, 0)

    @pl.when(w < NEXTRA)
    def _():
        d1 = pltpu.async_copy(ones_v, deg_s.at[sbuf.at[NR, 0]], sem, add=True)
        d2 = pltpu.async_copy(ones_v, deg_d.at[dbuf.at[NR, 0]], sem, add=True)
        d1.wait()
        d2.wait()

    plsc.subcore_barrier()

    @pl.when(sid == 0)
    def _():
        pltpu.sync_copy(deg_s, out_hbm.at[cid, 0])
        pltpu.sync_copy(deg_d, out_hbm.at[cid, 2])


def _deg(src3, dst3):
    call = pl.kernel(
        _deg_body,
        out_type=jax.ShapeDtypeStruct((NC, 4, N), _f32),
        mesh=_mesh(),
        scratch_types=[
            pltpu.VMEM((NR + 1, 1, CHUNK), _i32),
            pltpu.VMEM((NR + 1, 1, CHUNK), _i32),
            pltpu.VMEM((CHUNK,), _f32),
            pltpu.VMEM((1024,), _f32),
            pltpu.VMEM_SHARED((N,), _f32),
            pltpu.VMEM_SHARED((N,), _f32),
            pltpu.SemaphoreType.DMA,
        ],
    )
    return call(src3, dst3)


_NPAIRS = NR // 2


def _agg_body(xs_hbm, src3_hbm, dst3_hbm, out_hbm,
              dbuf, sidx0, sidx1, rows0, rows1, agg_sh, sem):
    cid = lax.axis_index("c")
    sid = lax.axis_index("s")
    w = cid * NS + sid
    nch = jnp.where(w < NEXTRA, NR + 1, NR)

    def zrow(r, carry):
        for j in range(D // LANES):
            rows0[r, pl.ds(j * LANES, LANES)] = jnp.zeros((LANES,), _f32)
        return carry

    lax.fori_loop(0, CHUNK, zrow, 0)

    @pl.when(sid < 10)
    def _():
        for k in range(7):
            pltpu.sync_copy(rows0, agg_sh.at[pl.ds(sid * 1000 + 128 * k, CHUNK)])
        pltpu.sync_copy(rows0, agg_sh.at[pl.ds(sid * 1000 + 872, CHUNK)])

    pltpu.sync_copy(dst3_hbm.at[pl.ds(w * NR, NR)], dbuf.at[pl.ds(0, NR)])

    @pl.when(w < NEXTRA)
    def _():
        pltpu.sync_copy(dst3_hbm.at[pl.ds(NW * NR + w, 1)], dbuf.at[pl.ds(NR, 1)])

    plsc.subcore_barrier()

    def sload(i, sidx):
        pltpu.sync_copy(src3_hbm.at[pl.ds(_row_of(w, i), 1)], sidx)

    def gather(sidx, rows):
        return pltpu.async_copy(xs_hbm.at[sidx.at[0, 0]], rows, sem)

    def gwait(rows):
        pltpu.make_async_copy(xs_hbm.at[sidx0.at[0, 0]], rows, sem).wait()

    def scatter(i, rows):
        pltpu.sync_copy(rows, agg_sh.at[dbuf.at[i, 0]], add=True)

    sload(0, sidx0)
    gather(sidx0, rows0)
    sload(1, sidx1)

    def pair(i2, carry):
        i = 2 * i2
        gwait(rows0)
        gather(sidx1, rows1)

        @pl.when(i + 2 < nch)
        def _():
            sload(i + 2, sidx0)

        scatter(i, rows0)
        gwait(rows1)

        @pl.when(i + 2 < nch)
        def _():
            gather(sidx0, rows0)

        @pl.when(i + 3 < nch)
        def _():
            sload(i + 3, sidx1)

        scatter(i + 1, rows1)
        return carry

    lax.fori_loop(0, _NPAIRS, pair, 0)

    @pl.when(nch == NR + 1)
    def _():
        gwait(rows0)
        scatter(NR, rows0)

    plsc.subcore_barrier()

    @pl.when(sid < 10)
    def _():
        pltpu.sync_copy(agg_sh.at[pl.ds(sid * 1000, 1000)],
                        out_hbm.at[cid, pl.ds(sid * 1000, 1000)])


def _agg(xs, src3, dst3):
    call = pl.kernel(
        _agg_body,
        out_type=jax.ShapeDtypeStruct((NC, N, D), _f32),
        mesh=_mesh(),
        scratch_types=[
            pltpu.VMEM((NR + 1, 1, CHUNK), _i32),
            pltpu.VMEM((1, 1, CHUNK), _i32),
            pltpu.VMEM((1, 1, CHUNK), _i32),
            pltpu.VMEM((CHUNK, D), _f32),
            pltpu.VMEM((CHUNK, D), _f32),
            pltpu.VMEM_SHARED((N, D), _f32),
            pltpu.SemaphoreType.DMA,
        ],
    )
    return call(xs, src3, dst3)


def _prep(degs, h):
    def body(degs_ref, h_ref, hs_ref, ns_ref, nd_ref):
        d = degs_ref[...]
        ns = lax.rsqrt(jnp.maximum(d[0, 0] + d[1, 0], 1.0))[:, None]
        nd = lax.rsqrt(jnp.maximum(d[0, 2] + d[1, 2], 1.0))[:, None]
        hs_ref[...] = h_ref[...] * ns
        ns_ref[...] = ns
        nd_ref[...] = nd

    return pl.pallas_call(
        body,
        out_shape=[
            jax.ShapeDtypeStruct((N, D), _f32),
            jax.ShapeDtypeStruct((N, 1), _f32),
            jax.ShapeDtypeStruct((N, 1), _f32),
        ],
    )(degs, h)


_GRID = 10
_BR = N // _GRID


def _layer(agg, nd, ns, W, b):
    def body(agg_ref, nd_ref, ns_ref, W_ref, b_ref, o_ref):
        a = (agg_ref[0] + agg_ref[1]) * nd_ref[...]
        y = jnp.dot(a, W_ref[...], preferred_element_type=_f32) + b_ref[...]
        o_ref[...] = jnp.maximum(y, 0.0) * ns_ref[...]

    return pl.pallas_call(
        body,
        grid=(_GRID,),
        in_specs=[
            pl.BlockSpec((NC, _BR, D), lambda i: (0, i, 0)),
            pl.BlockSpec((_BR, 1), lambda i: (i, 0)),
            pl.BlockSpec((_BR, 1), lambda i: (i, 0)),
            pl.BlockSpec((D, D), lambda i: (0, 0)),
            pl.BlockSpec((1, D), lambda i: (0, 0)),
        ],
        out_specs=pl.BlockSpec((_BR, D), lambda i: (i, 0)),
        out_shape=jax.ShapeDtypeStruct((N, D), _f32),
    )(agg, nd, ns, W, b.reshape(1, D))


def _out(agg, nd, W2, b2, Wc_p, bc_p):
    def body(agg_ref, nd_ref, W_ref, b_ref, Wc_ref, bc_ref, o_ref):
        a = (agg_ref[0] + agg_ref[1]) * nd_ref[...]
        y = jnp.dot(a, W_ref[...], preferred_element_type=_f32) + b_ref[...]
        y = jnp.maximum(y, 0.0)
        o_ref[...] = jnp.dot(y, Wc_ref[...], preferred_element_type=_f32) + bc_ref[...]

    return pl.pallas_call(
        body,
        grid=(_GRID,),
        in_specs=[
            pl.BlockSpec((NC, _BR, D), lambda i: (0, i, 0)),
            pl.BlockSpec((_BR, 1), lambda i: (i, 0)),
            pl.BlockSpec((D, D), lambda i: (0, 0)),
            pl.BlockSpec((1, D), lambda i: (0, 0)),
            pl.BlockSpec((D, D), lambda i: (0, 0)),
            pl.BlockSpec((1, D), lambda i: (0, 0)),
        ],
        out_specs=pl.BlockSpec((_BR, D), lambda i: (i, 0)),
        out_shape=jax.ShapeDtypeStruct((N, D), _f32),
    )(agg, nd, W2, b2.reshape(1, D), Wc_p, bc_p.reshape(1, D))


def kernel(h, edge_index, W1, b1, W2, b2, Wc, bc):
    src3 = edge_index[0].astype(_i32).reshape(NROWS, 1, CHUNK)
    dst3 = edge_index[1].astype(_i32).reshape(NROWS, 1, CHUNK)

    degs = _deg(src3, dst3)
    hs1, ns, nd = _prep(degs, h)
    agg1 = _agg(hs1, src3, dst3)
    hs2 = _layer(agg1, nd, ns, W1, b1)
    agg2 = _agg(hs2, src3, dst3)

    Wc_p = jnp.zeros((D, D), _f32).at[:, :NCLS].set(Wc)
    bc_p = jnp.zeros((D,), _f32).at[:NCLS].set(bc)
    outp = _out(agg2, nd, W2, b2, Wc_p, bc_p)
    return outp[:, :NCLS]
